# Initial kernel scaffold; baseline (speedup 1.0000x reference)
import jax
import jax.numpy as jnp
from jax.experimental import pallas as pl


def kernel(x, edge_index, batch, W1, b1, W2, b2, W3, b3, Wout, bout):
    raise NotImplementedError("write your pallas kernel here")


# speedup vs baseline: 13.7507x; 13.7507x over previous
import jax
import jax.numpy as jnp
from jax import lax
from jax.experimental import pallas as pl
from jax.experimental.pallas import tpu as pltpu
from jax.experimental.pallas import tpu_sc as plsc

V = 10000
E = 320000
D = 128
G = 128
NC = 2
NS = 16
NW = NC * NS
K = 128
NB = E // K
NB_LO = NB // NW
NB_REM = NB % NW
V_PAD = 10240
RPT = V_PAD // NS

_f32 = jnp.float32
_mesh = plsc.VectorSubcoreMesh(core_axis_name="c", subcore_axis_name="s",
                               num_cores=NC, num_subcores=NS)


def _sc_deg_body(edge_hbm, ones_hbm, zdeg_hbm, out_hbm, accd, didx, ones_v):
    c = lax.axis_index("c")
    s = lax.axis_index("s")
    wid = s * NC + c
    pltpu.sync_copy(zdeg_hbm, accd.at[pl.ds(s * RPT, RPT)])
    pltpu.sync_copy(ones_hbm, ones_v)
    plsc.subcore_barrier()
    nb = NB_LO + jnp.where(wid < NB_REM, 1, 0)

    def body(i, carry):
        off = (wid + NW * i) * K
        pltpu.sync_copy(edge_hbm.at[1, pl.ds(off, K)], didx)
        pltpu.sync_copy(ones_v, accd.at[didx], add=True)
        return carry

    lax.fori_loop(0, nb, body, 0)
    plsc.subcore_barrier()
    pltpu.sync_copy(accd.at[pl.ds(s * RPT, RPT)],
                    out_hbm.at[c, pl.ds(s * RPT, RPT)])


_sc_deg = pl.kernel(
    _sc_deg_body,
    out_type=jax.ShapeDtypeStruct((NC, V_PAD, 16), _f32),
    mesh=_mesh,
    scratch_types=[
        pltpu.VMEM_SHARED((V_PAD, 16), _f32),
        pltpu.VMEM((K,), jnp.int32),
        pltpu.VMEM((K, 16), _f32),
    ],
)


def _sc_edge_body(y_hbm, edge_hbm, zrows_hbm, out_hbm, acc, sidx, didx, rows,
                  gsem):
    c = lax.axis_index("c")
    s = lax.axis_index("s")
    wid = s * NC + c
    pltpu.sync_copy(zrows_hbm, acc.at[pl.ds(s * RPT, RPT)])
    plsc.subcore_barrier()
    nb = NB_LO + jnp.where(wid < NB_REM, 1, 0)

    def body(i, carry):
        off = (wid + NW * i) * K
        pltpu.sync_copy(edge_hbm.at[0, pl.ds(off, K)], sidx)
        pltpu.sync_copy(edge_hbm.at[1, pl.ds(off, K)], didx)
        pltpu.async_copy(y_hbm.at[sidx], rows, gsem).wait()
        pltpu.sync_copy(rows, acc.at[didx], add=True)
        return carry

    lax.fori_loop(0, nb, body, 0)
    plsc.subcore_barrier()
    pltpu.sync_copy(acc.at[pl.ds(s * RPT, RPT)],
                    out_hbm.at[c, pl.ds(s * RPT, RPT)])


_sc_edge = pl.kernel(
    _sc_edge_body,
    out_type=jax.ShapeDtypeStruct((NC, V_PAD, D), _f32),
    mesh=_mesh,
    scratch_types=[
        pltpu.VMEM_SHARED((V_PAD, D), _f32),
        pltpu.VMEM((K,), jnp.int32),
        pltpu.VMEM((K,), jnp.int32),
        pltpu.VMEM((K, D), _f32),
        pltpu.SemaphoreType.DMA,
    ],
)


def _tc1_body(x_ref, w_ref, degp_ref, y_ref, dinv_ref):
    deg = degp_ref[0, :V, 0:1] + degp_ref[1, :V, 0:1] + 1.0
    dinv = lax.rsqrt(deg)
    dinv_ref[...] = dinv
    xw = jnp.dot(x_ref[...], w_ref[...], preferred_element_type=_f32)
    y_ref[...] = xw * dinv


def _tc1(x, w1, degp):
    return pl.pallas_call(
        _tc1_body,
        out_shape=[
            jax.ShapeDtypeStruct((V, D), _f32),
            jax.ShapeDtypeStruct((V, 1), _f32),
        ],
    )(x, w1, degp)


def _tc_mid_body(sp_ref, y_ref, dinv_ref, b_ref, w_ref, yout_ref):
    dinv = dinv_ref[...]
    pre = dinv * (sp_ref[0, :V, :] + sp_ref[1, :V, :] + y_ref[...]) + b_ref[...]
    h = jnp.maximum(pre, 0.0)
    yout_ref[...] = jnp.dot(h, w_ref[...], preferred_element_type=_f32) * dinv


def _tc_mid(sp, y, dinv, b, w_next):
    return pl.pallas_call(
        _tc_mid_body,
        out_shape=jax.ShapeDtypeStruct((V, D), _f32),
    )(sp, y, dinv, b, w_next)


def _tc_fin_body(sp_ref, y_ref, dinv_ref, b_ref, batch_ref, wout_ref,
                 bout_ref, out_ref):
    dinv = dinv_ref[...]
    pre = dinv * (sp_ref[0, :V, :] + sp_ref[1, :V, :] + y_ref[...]) + b_ref[...]
    h = jnp.maximum(pre, 0.0)
    gids = lax.broadcasted_iota(jnp.int32, (G, 1), 0)
    onehot_t = (batch_ref[...] == gids).astype(_f32)
    pooled = jnp.dot(onehot_t, h, preferred_element_type=_f32)
    out_ref[...] = (jnp.dot(pooled, wout_ref[...], preferred_element_type=_f32)
                    + bout_ref[...])


def _tc_fin(sp, y, dinv, b, batch2d, wout, bout):
    n_class = wout.shape[1]
    return pl.pallas_call(
        _tc_fin_body,
        out_shape=jax.ShapeDtypeStruct((G, n_class), _f32),
    )(sp, y, dinv, b, batch2d, wout, bout)


def kernel(x, edge_index, batch, W1, b1, W2, b2, W3, b3, Wout, bout):
    edge = edge_index.astype(jnp.int32)
    ones16 = jnp.ones((K, 16), _f32)
    zdeg = jnp.zeros((RPT, 16), _f32)
    zrows = jnp.zeros((RPT, D), _f32)

    degp = _sc_deg(edge, ones16, zdeg)
    y1, dinv = _tc1(x, W1, degp)
    sp1 = _sc_edge(y1, edge, zrows)
    y2 = _tc_mid(sp1, y1, dinv, b1.reshape(1, D), W2)
    sp2 = _sc_edge(y2, edge, zrows)
    y3 = _tc_mid(sp2, y2, dinv, b2.reshape(1, D), W3)
    sp3 = _sc_edge(y3, edge, zrows)
    out = _tc_fin(sp3, y3, dinv, b3.reshape(1, D), batch.reshape(1, V),
                  Wout, bout.reshape(1, -1))
    return out
